# baseline (speedup 1.0000x reference)
import functools

import jax
import jax.numpy as jnp
from jax import lax
from jax.experimental import pallas as pl
from jax.experimental.pallas import tpu as pltpu
from jax.experimental.pallas import tpu_sc as plsc

T = 2048
D = 1024
FF = 512
E = 8
SHARED_FF2 = 2048
TILE = 128
PAD = 5120
NT = PAD // TILE
NB = 16
NW = 32
TPW = T // NW


def _router_body(x_ref, gate_ref, meta_ref, te_ref, wrow1_ref, wrow2_ref):
    x = x_ref[...]
    logits = jnp.dot(x, gate_ref[...].T, preferred_element_type=jnp.float32)
    probs = jax.nn.softmax(logits, axis=-1)
    iota_e = lax.broadcasted_iota(jnp.int32, probs.shape, 1)
    m1 = jnp.max(probs, axis=1, keepdims=True)
    idx1 = jnp.min(jnp.where(probs == m1, iota_e, E), axis=1, keepdims=True)
    oh1 = iota_e == idx1
    masked = jnp.where(oh1, -1.0, probs)
    m2 = jnp.max(masked, axis=1, keepdims=True)
    idx2 = jnp.min(jnp.where(masked == m2, iota_e, E), axis=1, keepdims=True)
    oh2 = iota_e == idx2
    wsum = m1 + m2

    sel = (oh1 | oh2).astype(jnp.float32)
    r_i = lax.broadcasted_iota(jnp.int32, (TILE, TILE), 0)
    c_i = lax.broadcasted_iota(jnp.int32, (TILE, TILE), 1)
    tril = jnp.where(r_i > c_i, 1.0, 0.0)
    off = jnp.zeros((1, E), jnp.float32)
    ranks = []
    for b in range(NB):
        sb = sel[b * TILE:(b + 1) * TILE, :]
        ranks.append(jnp.dot(tril, sb, preferred_element_type=jnp.float32)
                     + off)
        off = off + jnp.sum(sb, axis=0, keepdims=True)
    rank = jnp.concatenate(ranks, axis=0)
    counts = off
    cpad = jnp.floor((counts + (TILE - 1.0)) * (1.0 / TILE)) * TILE
    ru = lax.broadcasted_iota(jnp.int32, (E, E), 0)
    cu = lax.broadcasted_iota(jnp.int32, (E, E), 1)
    upper = jnp.where(ru < cu, 1.0, 0.0)
    base = jnp.dot(cpad, upper, preferred_element_type=jnp.float32)
    pos = base + rank
    pos1 = jnp.sum(jnp.where(oh1, pos, 0.0), axis=1, keepdims=True)
    pos2 = jnp.sum(jnp.where(oh2, pos, 0.0), axis=1, keepdims=True)

    col = lax.broadcasted_iota(jnp.int32, (T, 8), 1)
    meta = jnp.where(col == 0, pos1, 0.0)
    meta = jnp.where(col == 1, pos2, meta)
    meta_ref[...] = meta
    wrow1_ref[...] = jnp.broadcast_to(m1 / wsum, (T, 128))
    wrow2_ref[...] = jnp.broadcast_to(m2 / wsum, (T, 128))

    incl = base + cpad
    jv = lax.broadcasted_iota(jnp.int32, (64, 1), 0).astype(jnp.float32) * TILE
    teacc = jnp.zeros((64, 1), jnp.float32)
    for e in range(E):
        teacc = teacc + jnp.where(jv >= incl[:, e:e + 1], 1.0, 0.0)
    te_ref[...] = jnp.minimum(teacc, E - 1.0)


def _router(x, gate_w):
    return pl.pallas_call(
        _router_body,
        in_specs=[
            pl.BlockSpec((T, D), lambda: (0, 0)),
            pl.BlockSpec((E, D), lambda: (0, 0)),
        ],
        out_specs=[
            pl.BlockSpec((T, 8), lambda: (0, 0)),
            pl.BlockSpec((64, 1), lambda: (0, 0)),
            pl.BlockSpec((T, 128), lambda: (0, 0)),
            pl.BlockSpec((T, 128), lambda: (0, 0)),
        ],
        out_shape=[
            jax.ShapeDtypeStruct((T, 8), jnp.float32),
            jax.ShapeDtypeStruct((64, 1), jnp.float32),
            jax.ShapeDtypeStruct((T, 128), jnp.float32),
            jax.ShapeDtypeStruct((T, 128), jnp.float32),
        ],
    )(x, gate_w)


def _dispatch_body(pos1_hbm, pos2_hbm, wrow1_hbm, wrow2_hbm, x_hbm,
                   xs_hbm, ws_hbm, rows_v, wr1_v, wr2_v, idx1_v, idx2_v,
                   sem0, sem1, sem2, sem3):
    wid = lax.axis_index("s") * 2 + lax.axis_index("c")
    base = wid * TPW
    pltpu.sync_copy(pos1_hbm.at[pl.ds(base, TPW)], idx1_v)
    pltpu.sync_copy(pos2_hbm.at[pl.ds(base, TPW)], idx2_v)
    pltpu.sync_copy(x_hbm.at[pl.ds(base, TPW)], rows_v)
    pltpu.sync_copy(wrow1_hbm.at[pl.ds(base, TPW)], wr1_v)
    pltpu.sync_copy(wrow2_hbm.at[pl.ds(base, TPW)], wr2_v)
    cp1 = pltpu.make_async_copy(rows_v, xs_hbm.at[idx1_v], sem0)
    cp2 = pltpu.make_async_copy(rows_v, xs_hbm.at[idx2_v], sem1)
    cp3 = pltpu.make_async_copy(wr1_v, ws_hbm.at[idx1_v], sem2)
    cp4 = pltpu.make_async_copy(wr2_v, ws_hbm.at[idx2_v], sem3)
    cp1.start()
    cp2.start()
    cp3.start()
    cp4.start()
    cp1.wait()
    cp2.wait()
    cp3.wait()
    cp4.wait()


def _dispatch(pos1, pos2, wrow1, wrow2, x):
    mesh = plsc.VectorSubcoreMesh(core_axis_name="c", subcore_axis_name="s")
    fn = pl.kernel(
        _dispatch_body,
        mesh=mesh,
        out_type=[
            jax.ShapeDtypeStruct((PAD, D), jnp.float32),
            jax.ShapeDtypeStruct((PAD, 128), jnp.float32),
        ],
        scratch_types=[
            pltpu.VMEM((TPW, D), jnp.float32),
            pltpu.VMEM((TPW, 128), jnp.float32),
            pltpu.VMEM((TPW, 128), jnp.float32),
            pltpu.VMEM((TPW,), jnp.int32),
            pltpu.VMEM((TPW,), jnp.int32),
            pltpu.SemaphoreType.DMA,
            pltpu.SemaphoreType.DMA,
            pltpu.SemaphoreType.DMA,
            pltpu.SemaphoreType.DMA,
        ],
    )
    return fn(pos1, pos2, wrow1, wrow2, x)


def _group_mm_body(te_ref, xs_ref, ws_ref, wgu_ref, wd_ref, ys_ref):
    xb = xs_ref[...].astype(jnp.bfloat16)
    gu = jnp.dot(xb, wgu_ref[0].T, preferred_element_type=jnp.float32)
    g, u = gu[:, :FF], gu[:, FF:]
    h = (g * jax.nn.sigmoid(g) * u).astype(jnp.bfloat16)
    y = jnp.dot(h, wd_ref[0].T, preferred_element_type=jnp.float32)
    ys_ref[...] = y * ws_ref[:, :1]


def _group_mm(te, xs, ws, wgu_bf, wd_bf):
    grid_spec = pltpu.PrefetchScalarGridSpec(
        num_scalar_prefetch=1,
        grid=(NT,),
        in_specs=[
            pl.BlockSpec((TILE, D), lambda i, te: (i, 0)),
            pl.BlockSpec((TILE, 128), lambda i, te: (i, 0)),
            pl.BlockSpec((1, 2 * FF, D), lambda i, te: (te[i], 0, 0)),
            pl.BlockSpec((1, D, FF), lambda i, te: (te[i], 0, 0)),
        ],
        out_specs=pl.BlockSpec((TILE, D), lambda i, te: (i, 0)),
    )
    return pl.pallas_call(
        _group_mm_body,
        grid_spec=grid_spec,
        out_shape=jax.ShapeDtypeStruct((PAD, D), jnp.float32),
    )(te, xs, ws, wgu_bf, wd_bf)


def _shared_body(x_ref, wsg_ref, wsd_ref, out_ref):
    xb = x_ref[...].astype(jnp.bfloat16)
    sgu = jnp.dot(xb, wsg_ref[...].T, preferred_element_type=jnp.float32)
    sg, su = sgu[:, :SHARED_FF2 // 2], sgu[:, SHARED_FF2 // 2:]
    sh = (sg * jax.nn.sigmoid(sg) * su).astype(jnp.bfloat16)
    out_ref[...] = jnp.dot(sh, wsd_ref[...].T,
                           preferred_element_type=jnp.float32)


def _shared(x, wsg_bf, wsd_bf):
    TM = 256
    return pl.pallas_call(
        _shared_body,
        grid=(T // TM,),
        in_specs=[
            pl.BlockSpec((TM, D), lambda i: (i, 0)),
            pl.BlockSpec((SHARED_FF2, D), lambda i: (0, 0)),
            pl.BlockSpec((D, SHARED_FF2 // 2), lambda i: (0, 0)),
        ],
        out_specs=pl.BlockSpec((TM, D), lambda i: (i, 0)),
        out_shape=jax.ShapeDtypeStruct((T, D), jnp.float32),
    )(x, wsg_bf, wsd_bf)


def _combine_body(pos1_hbm, pos2_hbm, ys_hbm, sh_hbm, out_hbm,
                  p1_v, p2_v, idx1c_v, idx2c_v, r1_v, r2_v, sh_v, out_v,
                  sem0, sem1):
    wid = lax.axis_index("s") * 2 + lax.axis_index("c")
    base = wid * TPW
    pltpu.sync_copy(pos1_hbm.at[pl.ds(base, TPW)], p1_v)
    pltpu.sync_copy(pos2_hbm.at[pl.ds(base, TPW)], p2_v)

    def chunk(c, carry):
        idx1c_v[...] = p1_v[pl.ds(16 * c, 16)]
        idx2c_v[...] = p2_v[pl.ds(16 * c, 16)]
        cp1 = pltpu.make_async_copy(ys_hbm.at[idx1c_v], r1_v, sem0)
        cp2 = pltpu.make_async_copy(ys_hbm.at[idx2c_v], r2_v, sem1)
        cp1.start()
        cp2.start()
        pltpu.sync_copy(sh_hbm.at[pl.ds(base + 16 * c, 16)], sh_v)
        cp1.wait()
        cp2.wait()

        def row(r, carry2):
            def col(k, carry3):
                sl = pl.ds(16 * k, 16)
                out_v[r, sl] = r1_v[r, sl] + r2_v[r, sl] + sh_v[r, sl]
                return carry3

            return lax.fori_loop(0, D // 16, col, carry2)

        lax.fori_loop(0, 16, row, 0)
        pltpu.sync_copy(out_v, out_hbm.at[pl.ds(base + 16 * c, 16)])
        return carry

    lax.fori_loop(0, TPW // 16, chunk, 0)


def _combine(pos1, pos2, ys, sh):
    mesh = plsc.VectorSubcoreMesh(core_axis_name="c", subcore_axis_name="s")
    fn = pl.kernel(
        _combine_body,
        mesh=mesh,
        out_type=jax.ShapeDtypeStruct((T, D), jnp.float32),
        scratch_types=[
            pltpu.VMEM((TPW,), jnp.int32),
            pltpu.VMEM((TPW,), jnp.int32),
            pltpu.VMEM((16,), jnp.int32),
            pltpu.VMEM((16,), jnp.int32),
            pltpu.VMEM((16, D), jnp.float32),
            pltpu.VMEM((16, D), jnp.float32),
            pltpu.VMEM((16, D), jnp.float32),
            pltpu.VMEM((16, D), jnp.float32),
            pltpu.SemaphoreType.DMA,
            pltpu.SemaphoreType.DMA,
        ],
    )
    return fn(pos1, pos2, ys, sh)


def kernel(hidden_states, gate_w, w_gate_up, w_down, ws_gate_up, ws_down):
    b, s, d = hidden_states.shape
    x = hidden_states.reshape(-1, d)
    wgu_bf = w_gate_up.astype(jnp.bfloat16)
    wd_bf = w_down.astype(jnp.bfloat16)
    wsg_bf = ws_gate_up.astype(jnp.bfloat16)
    wsd_bf = ws_down.astype(jnp.bfloat16)

    meta, te_f, wrow1, wrow2 = _router(x, gate_w)
    pos1 = meta[:, 0].astype(jnp.int32)
    pos2 = meta[:, 1].astype(jnp.int32)
    te = te_f.reshape(64)[:NT].astype(jnp.int32)
    xs, ws = _dispatch(pos1, pos2, wrow1, wrow2, x)
    sh = _shared(x, wsg_bf, wsd_bf)
    ys = _group_mm(te, xs, ws, wgu_bf, wd_bf)
    out = _combine(pos1, pos2, ys, sh)
    return out.reshape(b, s, d)


# speedup vs baseline: 1.2096x; 1.2096x over previous
import functools

import jax
import jax.numpy as jnp
from jax import lax
from jax.experimental import pallas as pl
from jax.experimental.pallas import tpu as pltpu
from jax.experimental.pallas import tpu_sc as plsc

T = 2048
D = 1024
DH = 512
FF = 512
E = 8
SHARED_FF2 = 2048
TILE = 128
PAD = 5120
NT = PAD // TILE
NB = 16
NW = 32
TPW = T // NW

def _pack_halves(y_f32):
    yb = y_f32.astype(jnp.bfloat16)
    a = lax.bitcast_convert_type(yb[:, :DH], jnp.uint16).astype(jnp.uint32)
    bhi = lax.bitcast_convert_type(yb[:, DH:], jnp.uint16).astype(jnp.uint32)
    return lax.bitcast_convert_type(a | (bhi << 16), jnp.int32)


def _unpack_halves_bf16(w_i32):
    lo = lax.bitcast_convert_type(w_i32 << 16, jnp.float32)
    hi = lax.bitcast_convert_type(w_i32 & jnp.int32(-65536), jnp.float32)
    return lo.astype(jnp.bfloat16), hi.astype(jnp.bfloat16)


def _router_body(x_ref, gate_ref, meta_ref, te_ref, wrow1_ref, wrow2_ref,
                 xp_ref):
    x = x_ref[...]
    logits = jnp.dot(x, gate_ref[...].T, preferred_element_type=jnp.float32)
    probs = jax.nn.softmax(logits, axis=-1)
    iota_e = lax.broadcasted_iota(jnp.int32, probs.shape, 1)
    m1 = jnp.max(probs, axis=1, keepdims=True)
    idx1 = jnp.min(jnp.where(probs == m1, iota_e, E), axis=1, keepdims=True)
    oh1 = iota_e == idx1
    masked = jnp.where(oh1, -1.0, probs)
    m2 = jnp.max(masked, axis=1, keepdims=True)
    idx2 = jnp.min(jnp.where(masked == m2, iota_e, E), axis=1, keepdims=True)
    oh2 = iota_e == idx2
    wsum = m1 + m2

    sel = (oh1 | oh2).astype(jnp.float32)
    r_i = lax.broadcasted_iota(jnp.int32, (TILE, TILE), 0)
    c_i = lax.broadcasted_iota(jnp.int32, (TILE, TILE), 1)
    tril = jnp.where(r_i > c_i, 1.0, 0.0)
    off = jnp.zeros((1, E), jnp.float32)
    ranks = []
    for b in range(NB):
        sb = sel[b * TILE:(b + 1) * TILE, :]
        ranks.append(jnp.dot(tril, sb, preferred_element_type=jnp.float32)
                     + off)
        off = off + jnp.sum(sb, axis=0, keepdims=True)
    rank = jnp.concatenate(ranks, axis=0)
    counts = off
    cpad = jnp.floor((counts + (TILE - 1.0)) * (1.0 / TILE)) * TILE
    ru = lax.broadcasted_iota(jnp.int32, (E, E), 0)
    cu = lax.broadcasted_iota(jnp.int32, (E, E), 1)
    upper = jnp.where(ru < cu, 1.0, 0.0)
    base = jnp.dot(cpad, upper, preferred_element_type=jnp.float32)
    pos = base + rank
    pos1 = jnp.sum(jnp.where(oh1, pos, 0.0), axis=1, keepdims=True)
    pos2 = jnp.sum(jnp.where(oh2, pos, 0.0), axis=1, keepdims=True)

    col = lax.broadcasted_iota(jnp.int32, (T, 8), 1)
    meta = jnp.where(col == 0, pos1, 0.0)
    meta = jnp.where(col == 1, pos2, meta)
    meta_ref[...] = meta
    wrow1_ref[...] = jnp.broadcast_to(m1 / wsum, (T, 128))
    wrow2_ref[...] = jnp.broadcast_to(m2 / wsum, (T, 128))
    xp_ref[...] = _pack_halves(x)

    incl = base + cpad
    jv = lax.broadcasted_iota(jnp.int32, (64, 1), 0).astype(jnp.float32) * TILE
    teacc = jnp.zeros((64, 1), jnp.float32)
    for e in range(E):
        teacc = teacc + jnp.where(jv >= incl[:, e:e + 1], 1.0, 0.0)
    te_ref[...] = jnp.minimum(teacc, E - 1.0)


def _router(x, gate_w):
    return pl.pallas_call(
        _router_body,
        in_specs=[
            pl.BlockSpec((T, D), lambda: (0, 0)),
            pl.BlockSpec((E, D), lambda: (0, 0)),
        ],
        out_specs=[
            pl.BlockSpec((T, 8), lambda: (0, 0)),
            pl.BlockSpec((64, 1), lambda: (0, 0)),
            pl.BlockSpec((T, 128), lambda: (0, 0)),
            pl.BlockSpec((T, 128), lambda: (0, 0)),
            pl.BlockSpec((T, DH), lambda: (0, 0)),
        ],
        out_shape=[
            jax.ShapeDtypeStruct((T, 8), jnp.float32),
            jax.ShapeDtypeStruct((64, 1), jnp.float32),
            jax.ShapeDtypeStruct((T, 128), jnp.float32),
            jax.ShapeDtypeStruct((T, 128), jnp.float32),
            jax.ShapeDtypeStruct((T, DH), jnp.int32),
        ],
    )(x, gate_w)


def _dispatch_body(pos1_hbm, pos2_hbm, wrow1_hbm, wrow2_hbm, xp_hbm,
                   xs_hbm, ws_hbm, rows_v, wr1_v, wr2_v, idx1_v, idx2_v,
                   sem0, sem1, sem2, sem3):
    wid = lax.axis_index("s") * 2 + lax.axis_index("c")
    base = wid * TPW
    pltpu.sync_copy(pos1_hbm.at[pl.ds(base, TPW)], idx1_v)
    pltpu.sync_copy(pos2_hbm.at[pl.ds(base, TPW)], idx2_v)
    pltpu.sync_copy(xp_hbm.at[pl.ds(base, TPW)], rows_v)
    pltpu.sync_copy(wrow1_hbm.at[pl.ds(base, TPW)], wr1_v)
    pltpu.sync_copy(wrow2_hbm.at[pl.ds(base, TPW)], wr2_v)
    cp1 = pltpu.make_async_copy(rows_v, xs_hbm.at[idx1_v], sem0)
    cp2 = pltpu.make_async_copy(rows_v, xs_hbm.at[idx2_v], sem1)
    cp3 = pltpu.make_async_copy(wr1_v, ws_hbm.at[idx1_v], sem2)
    cp4 = pltpu.make_async_copy(wr2_v, ws_hbm.at[idx2_v], sem3)
    cp1.start()
    cp2.start()
    cp3.start()
    cp4.start()
    cp1.wait()
    cp2.wait()
    cp3.wait()
    cp4.wait()


def _dispatch(pos1, pos2, wrow1, wrow2, xp):
    mesh = plsc.VectorSubcoreMesh(core_axis_name="c", subcore_axis_name="s")
    fn = pl.kernel(
        _dispatch_body,
        mesh=mesh,
        out_type=[
            jax.ShapeDtypeStruct((PAD, DH), jnp.int32),
            jax.ShapeDtypeStruct((PAD, 128), jnp.float32),
        ],
        scratch_types=[
            pltpu.VMEM((TPW, DH), jnp.int32),
            pltpu.VMEM((TPW, 128), jnp.float32),
            pltpu.VMEM((TPW, 128), jnp.float32),
            pltpu.VMEM((TPW,), jnp.int32),
            pltpu.VMEM((TPW,), jnp.int32),
            pltpu.SemaphoreType.DMA,
            pltpu.SemaphoreType.DMA,
            pltpu.SemaphoreType.DMA,
            pltpu.SemaphoreType.DMA,
        ],
    )
    return fn(pos1, pos2, wrow1, wrow2, xp)


def _group_mm_body(te_ref, xs_ref, ws_ref, wgu_ref, wd_ref, ys_ref,
                   wgu_bf, wd_bf):
    i = pl.program_id(0)
    e_now = te_ref[i]
    e_prev = te_ref[jnp.maximum(i - 1, 0)]

    @pl.when(jnp.logical_or(i == 0, e_now != e_prev))
    def _cast():
        wgu_bf[...] = wgu_ref[0].astype(jnp.bfloat16)
        wd_bf[...] = wd_ref[0].astype(jnp.bfloat16)

    a, bh = _unpack_halves_bf16(xs_ref[...])
    gu = (jnp.dot(a, wgu_bf[:, :DH].T, preferred_element_type=jnp.float32)
          + jnp.dot(bh, wgu_bf[:, DH:].T, preferred_element_type=jnp.float32))
    g, u = gu[:, :FF], gu[:, FF:]
    h = (g * jax.nn.sigmoid(g) * u).astype(jnp.bfloat16)
    y = jnp.dot(h, wd_bf[...].T, preferred_element_type=jnp.float32)
    ys_ref[...] = _pack_halves(y * ws_ref[:, :1])


def _group_mm(te, xs, ws, wgu, wd):
    grid_spec = pltpu.PrefetchScalarGridSpec(
        num_scalar_prefetch=1,
        grid=(NT,),
        in_specs=[
            pl.BlockSpec((TILE, DH), lambda i, te: (i, 0)),
            pl.BlockSpec((TILE, 128), lambda i, te: (i, 0)),
            pl.BlockSpec((1, 2 * FF, D), lambda i, te: (te[i], 0, 0)),
            pl.BlockSpec((1, D, FF), lambda i, te: (te[i], 0, 0)),
        ],
        out_specs=pl.BlockSpec((TILE, DH), lambda i, te: (i, 0)),
        scratch_shapes=[
            pltpu.VMEM((2 * FF, D), jnp.bfloat16),
            pltpu.VMEM((D, FF), jnp.bfloat16),
        ],
    )
    return pl.pallas_call(
        _group_mm_body,
        grid_spec=grid_spec,
        out_shape=jax.ShapeDtypeStruct((PAD, DH), jnp.int32),
    )(te, xs, ws, wgu, wd)


def _shared_body(x_ref, wsg_ref, wsd_ref, out_ref, wsg_bf, wsd_bf):
    @pl.when(pl.program_id(0) == 0)
    def _cast():
        wsg_bf[...] = wsg_ref[...].astype(jnp.bfloat16)
        wsd_bf[...] = wsd_ref[...].astype(jnp.bfloat16)

    xb = x_ref[...].astype(jnp.bfloat16)
    sgu = jnp.dot(xb, wsg_bf[...].T, preferred_element_type=jnp.float32)
    sg, su = sgu[:, :SHARED_FF2 // 2], sgu[:, SHARED_FF2 // 2:]
    sh = (sg * jax.nn.sigmoid(sg) * su).astype(jnp.bfloat16)
    out_ref[...] = _pack_halves(
        jnp.dot(sh, wsd_bf[...].T, preferred_element_type=jnp.float32))


def _shared(x, wsg, wsd):
    TM = 256
    return pl.pallas_call(
        _shared_body,
        grid=(T // TM,),
        in_specs=[
            pl.BlockSpec((TM, D), lambda i: (i, 0)),
            pl.BlockSpec((SHARED_FF2, D), lambda i: (0, 0)),
            pl.BlockSpec((D, SHARED_FF2 // 2), lambda i: (0, 0)),
        ],
        out_specs=pl.BlockSpec((TM, DH), lambda i: (i, 0)),
        out_shape=jax.ShapeDtypeStruct((T, DH), jnp.int32),
        scratch_shapes=[
            pltpu.VMEM((SHARED_FF2, D), jnp.bfloat16),
            pltpu.VMEM((D, SHARED_FF2 // 2), jnp.bfloat16),
        ],
    )(x, wsg, wsd)


def _combine_body(pos1_hbm, pos2_hbm, ys_hbm, sh_hbm, out_hbm,
                  p1_v, p2_v, idx1c_v, idx2c_v, r1_v, r2_v, sh_v, out_v,
                  sem0, sem1):
    wid = lax.axis_index("s") * 2 + lax.axis_index("c")
    base = wid * TPW
    pltpu.sync_copy(pos1_hbm.at[pl.ds(base, TPW)], p1_v)
    pltpu.sync_copy(pos2_hbm.at[pl.ds(base, TPW)], p2_v)
    himask = jnp.full((16,), -65536, jnp.int32)

    def chunk(c, carry):
        idx1c_v[...] = p1_v[pl.ds(16 * c, 16)]
        idx2c_v[...] = p2_v[pl.ds(16 * c, 16)]
        cp1 = pltpu.make_async_copy(ys_hbm.at[idx1c_v], r1_v, sem0)
        cp2 = pltpu.make_async_copy(ys_hbm.at[idx2c_v], r2_v, sem1)
        cp1.start()
        cp2.start()
        pltpu.sync_copy(sh_hbm.at[pl.ds(base + 16 * c, 16)], sh_v)
        cp1.wait()
        cp2.wait()

        def row(r, carry2):
            def col(k, carry3):
                sl = pl.ds(16 * k, 16)
                w1 = r1_v[r, sl]
                w2 = r2_v[r, sl]
                wsh = sh_v[r, sl]
                lo = (lax.bitcast_convert_type(w1 << 16, jnp.float32)
                      + lax.bitcast_convert_type(w2 << 16, jnp.float32)
                      + lax.bitcast_convert_type(wsh << 16, jnp.float32))
                hi = (lax.bitcast_convert_type(w1 & himask, jnp.float32)
                      + lax.bitcast_convert_type(w2 & himask, jnp.float32)
                      + lax.bitcast_convert_type(wsh & himask, jnp.float32))
                out_v[r, sl] = lo
                out_v[r, pl.ds(DH + 16 * k, 16)] = hi
                return carry3

            return lax.fori_loop(0, DH // 16, col, carry2)

        lax.fori_loop(0, 16, row, 0)
        pltpu.sync_copy(out_v, out_hbm.at[pl.ds(base + 16 * c, 16)])
        return carry

    lax.fori_loop(0, TPW // 16, chunk, 0)


def _combine(pos1, pos2, ys, sh):
    mesh = plsc.VectorSubcoreMesh(core_axis_name="c", subcore_axis_name="s")
    fn = pl.kernel(
        _combine_body,
        mesh=mesh,
        out_type=jax.ShapeDtypeStruct((T, D), jnp.float32),
        scratch_types=[
            pltpu.VMEM((TPW,), jnp.int32),
            pltpu.VMEM((TPW,), jnp.int32),
            pltpu.VMEM((16,), jnp.int32),
            pltpu.VMEM((16,), jnp.int32),
            pltpu.VMEM((16, DH), jnp.int32),
            pltpu.VMEM((16, DH), jnp.int32),
            pltpu.VMEM((16, DH), jnp.int32),
            pltpu.VMEM((16, D), jnp.float32),
            pltpu.SemaphoreType.DMA,
            pltpu.SemaphoreType.DMA,
        ],
    )
    return fn(pos1, pos2, ys, sh)


def kernel(hidden_states, gate_w, w_gate_up, w_down, ws_gate_up, ws_down):
    b, s, d = hidden_states.shape
    x = hidden_states.reshape(-1, d)

    meta, te_f, wrow1, wrow2, xp = _router(x, gate_w)
    pos1 = meta[:, 0].astype(jnp.int32)
    pos2 = meta[:, 1].astype(jnp.int32)
    te = te_f.reshape(64)[:NT].astype(jnp.int32)
    xs, ws = _dispatch(pos1, pos2, wrow1, wrow2, xp)
    sh = _shared(x, ws_gate_up, ws_down)
    ys = _group_mm(te, xs, ws, w_gate_up, w_down)
    out = _combine(pos1, pos2, ys, sh)
    return out.reshape(b, s, d)


# speedup vs baseline: 1.3957x; 1.1538x over previous
import functools

import jax
import jax.numpy as jnp
from jax import lax
from jax.experimental import pallas as pl
from jax.experimental.pallas import tpu as pltpu
from jax.experimental.pallas import tpu_sc as plsc

T = 2048
D = 1024
DH = 512
FF = 512
E = 8
SHARED_FF2 = 2048
TILE = 256
PAD = 6144
NT = PAD // TILE
BS = 128
NB = 16
NW = 32
TPW = T // NW

def _pack_halves(y_f32):
    yb = y_f32.astype(jnp.bfloat16)
    a = lax.bitcast_convert_type(yb[:, :DH], jnp.uint16).astype(jnp.uint32)
    bhi = lax.bitcast_convert_type(yb[:, DH:], jnp.uint16).astype(jnp.uint32)
    return lax.bitcast_convert_type(a | (bhi << 16), jnp.int32)


def _unpack_halves_bf16(w_i32):
    lo = lax.bitcast_convert_type(w_i32 << 16, jnp.float32)
    hi = lax.bitcast_convert_type(w_i32 & jnp.int32(-65536), jnp.float32)
    return lo.astype(jnp.bfloat16), hi.astype(jnp.bfloat16)


def _router_body(x_ref, gate_ref, meta_ref, te_ref, wrow1_ref, wrow2_ref,
                 xp_ref):
    x = x_ref[...]
    logits = jnp.dot(x, gate_ref[...].T, preferred_element_type=jnp.float32)
    probs = jax.nn.softmax(logits, axis=-1)
    iota_e = lax.broadcasted_iota(jnp.int32, probs.shape, 1)
    m1 = jnp.max(probs, axis=1, keepdims=True)
    idx1 = jnp.min(jnp.where(probs == m1, iota_e, E), axis=1, keepdims=True)
    oh1 = iota_e == idx1
    masked = jnp.where(oh1, -1.0, probs)
    m2 = jnp.max(masked, axis=1, keepdims=True)
    idx2 = jnp.min(jnp.where(masked == m2, iota_e, E), axis=1, keepdims=True)
    oh2 = iota_e == idx2
    wsum = m1 + m2

    sel = (oh1 | oh2).astype(jnp.float32)
    r_i = lax.broadcasted_iota(jnp.int32, (BS, BS), 0)
    c_i = lax.broadcasted_iota(jnp.int32, (BS, BS), 1)
    tril = jnp.where(r_i > c_i, 1.0, 0.0)
    off = jnp.zeros((1, E), jnp.float32)
    ranks = []
    for b in range(NB):
        sb = sel[b * BS:(b + 1) * BS, :]
        ranks.append(jnp.dot(tril, sb, preferred_element_type=jnp.float32)
                     + off)
        off = off + jnp.sum(sb, axis=0, keepdims=True)
    rank = jnp.concatenate(ranks, axis=0)
    counts = off
    cpad = jnp.floor((counts + (TILE - 1.0)) * (1.0 / TILE)) * TILE
    ru = lax.broadcasted_iota(jnp.int32, (E, E), 0)
    cu = lax.broadcasted_iota(jnp.int32, (E, E), 1)
    upper = jnp.where(ru < cu, 1.0, 0.0)
    base = jnp.dot(cpad, upper, preferred_element_type=jnp.float32)
    pos = base + rank
    pos1 = jnp.sum(jnp.where(oh1, pos, 0.0), axis=1, keepdims=True)
    pos2 = jnp.sum(jnp.where(oh2, pos, 0.0), axis=1, keepdims=True)

    col = lax.broadcasted_iota(jnp.int32, (T, 8), 1)
    meta = jnp.where(col == 0, pos1, 0.0)
    meta = jnp.where(col == 1, pos2, meta)
    meta_ref[...] = meta
    wrow1_ref[...] = jnp.broadcast_to(m1 / wsum, (T, 128))
    wrow2_ref[...] = jnp.broadcast_to(m2 / wsum, (T, 128))
    xp_ref[...] = _pack_halves(x)

    incl = base + cpad
    jv = lax.broadcasted_iota(jnp.int32, (64, 1), 0).astype(jnp.float32) * TILE
    teacc = jnp.zeros((64, 1), jnp.float32)
    for e in range(E):
        teacc = teacc + jnp.where(jv >= incl[:, e:e + 1], 1.0, 0.0)
    te_ref[...] = jnp.minimum(teacc, E - 1.0)


def _router(x, gate_w):
    return pl.pallas_call(
        _router_body,
        in_specs=[
            pl.BlockSpec((T, D), lambda: (0, 0)),
            pl.BlockSpec((E, D), lambda: (0, 0)),
        ],
        out_specs=[
            pl.BlockSpec((T, 8), lambda: (0, 0)),
            pl.BlockSpec((64, 1), lambda: (0, 0)),
            pl.BlockSpec((T, 128), lambda: (0, 0)),
            pl.BlockSpec((T, 128), lambda: (0, 0)),
            pl.BlockSpec((T, DH), lambda: (0, 0)),
        ],
        out_shape=[
            jax.ShapeDtypeStruct((T, 8), jnp.float32),
            jax.ShapeDtypeStruct((64, 1), jnp.float32),
            jax.ShapeDtypeStruct((T, 128), jnp.float32),
            jax.ShapeDtypeStruct((T, 128), jnp.float32),
            jax.ShapeDtypeStruct((T, DH), jnp.int32),
        ],
    )(x, gate_w)


def _dispatch_body(pos1_hbm, pos2_hbm, wrow1_hbm, wrow2_hbm, xp_hbm,
                   xs_hbm, ws_hbm, rows_v, wr1_v, wr2_v, idx1_v, idx2_v,
                   sem0, sem1, sem2, sem3):
    wid = lax.axis_index("s") * 2 + lax.axis_index("c")
    base = wid * TPW
    pltpu.sync_copy(pos1_hbm.at[pl.ds(base, TPW)], idx1_v)
    pltpu.sync_copy(pos2_hbm.at[pl.ds(base, TPW)], idx2_v)
    pltpu.sync_copy(xp_hbm.at[pl.ds(base, TPW)], rows_v)
    pltpu.sync_copy(wrow1_hbm.at[pl.ds(base, TPW)], wr1_v)
    pltpu.sync_copy(wrow2_hbm.at[pl.ds(base, TPW)], wr2_v)
    cp1 = pltpu.make_async_copy(rows_v, xs_hbm.at[idx1_v], sem0)
    cp2 = pltpu.make_async_copy(rows_v, xs_hbm.at[idx2_v], sem1)
    cp3 = pltpu.make_async_copy(wr1_v, ws_hbm.at[idx1_v], sem2)
    cp4 = pltpu.make_async_copy(wr2_v, ws_hbm.at[idx2_v], sem3)
    cp1.start()
    cp2.start()
    cp3.start()
    cp4.start()
    cp1.wait()
    cp2.wait()
    cp3.wait()
    cp4.wait()


def _dispatch(pos1, pos2, wrow1, wrow2, xp):
    mesh = plsc.VectorSubcoreMesh(core_axis_name="c", subcore_axis_name="s")
    fn = pl.kernel(
        _dispatch_body,
        mesh=mesh,
        out_type=[
            jax.ShapeDtypeStruct((PAD, DH), jnp.int32),
            jax.ShapeDtypeStruct((PAD, 128), jnp.float32),
        ],
        scratch_types=[
            pltpu.VMEM((TPW, DH), jnp.int32),
            pltpu.VMEM((TPW, 128), jnp.float32),
            pltpu.VMEM((TPW, 128), jnp.float32),
            pltpu.VMEM((TPW,), jnp.int32),
            pltpu.VMEM((TPW,), jnp.int32),
            pltpu.SemaphoreType.DMA,
            pltpu.SemaphoreType.DMA,
            pltpu.SemaphoreType.DMA,
            pltpu.SemaphoreType.DMA,
        ],
    )
    return fn(pos1, pos2, wrow1, wrow2, xp)


def _group_mm_body(te_ref, xs_ref, ws_ref, wgu_ref, wd_ref, ys_ref,
                   wgu_bf, wd_bf):
    i = pl.program_id(0)
    e_now = te_ref[i]
    e_prev = te_ref[jnp.maximum(i - 1, 0)]

    @pl.when(jnp.logical_or(i == 0, e_now != e_prev))
    def _cast():
        wgu_bf[...] = wgu_ref[0].astype(jnp.bfloat16)
        wd_bf[...] = wd_ref[0].astype(jnp.bfloat16)

    a, bh = _unpack_halves_bf16(xs_ref[...])
    gu = (jnp.dot(a, wgu_bf[:, :DH].T, preferred_element_type=jnp.float32)
          + jnp.dot(bh, wgu_bf[:, DH:].T, preferred_element_type=jnp.float32))
    g, u = gu[:, :FF], gu[:, FF:]
    h = (g * jax.nn.sigmoid(g) * u).astype(jnp.bfloat16)
    y = jnp.dot(h, wd_bf[...].T, preferred_element_type=jnp.float32)
    ys_ref[...] = _pack_halves(y * ws_ref[:, :1])


def _group_mm(te, xs, ws, wgu, wd):
    grid_spec = pltpu.PrefetchScalarGridSpec(
        num_scalar_prefetch=1,
        grid=(NT,),
        in_specs=[
            pl.BlockSpec((TILE, DH), lambda i, te: (i, 0)),
            pl.BlockSpec((TILE, 128), lambda i, te: (i, 0)),
            pl.BlockSpec((1, 2 * FF, D), lambda i, te: (te[i], 0, 0)),
            pl.BlockSpec((1, D, FF), lambda i, te: (te[i], 0, 0)),
        ],
        out_specs=pl.BlockSpec((TILE, DH), lambda i, te: (i, 0)),
        scratch_shapes=[
            pltpu.VMEM((2 * FF, D), jnp.bfloat16),
            pltpu.VMEM((D, FF), jnp.bfloat16),
        ],
    )
    return pl.pallas_call(
        _group_mm_body,
        grid_spec=grid_spec,
        out_shape=jax.ShapeDtypeStruct((PAD, DH), jnp.int32),
    )(te, xs, ws, wgu, wd)


def _shared_body(x_ref, wsg_ref, wsd_ref, out_ref, wsg_bf, wsd_bf):
    @pl.when(pl.program_id(0) == 0)
    def _cast():
        wsg_bf[...] = wsg_ref[...].astype(jnp.bfloat16)
        wsd_bf[...] = wsd_ref[...].astype(jnp.bfloat16)

    xb = x_ref[...].astype(jnp.bfloat16)
    sgu = jnp.dot(xb, wsg_bf[...].T, preferred_element_type=jnp.float32)
    sg, su = sgu[:, :SHARED_FF2 // 2], sgu[:, SHARED_FF2 // 2:]
    sh = (sg * jax.nn.sigmoid(sg) * su).astype(jnp.bfloat16)
    out_ref[...] = _pack_halves(
        jnp.dot(sh, wsd_bf[...].T, preferred_element_type=jnp.float32))


def _shared(x, wsg, wsd):
    TM = 256
    return pl.pallas_call(
        _shared_body,
        grid=(T // TM,),
        in_specs=[
            pl.BlockSpec((TM, D), lambda i: (i, 0)),
            pl.BlockSpec((SHARED_FF2, D), lambda i: (0, 0)),
            pl.BlockSpec((D, SHARED_FF2 // 2), lambda i: (0, 0)),
        ],
        out_specs=pl.BlockSpec((TM, DH), lambda i: (i, 0)),
        out_shape=jax.ShapeDtypeStruct((T, DH), jnp.int32),
        scratch_shapes=[
            pltpu.VMEM((SHARED_FF2, D), jnp.bfloat16),
            pltpu.VMEM((D, SHARED_FF2 // 2), jnp.bfloat16),
        ],
    )(x, wsg, wsd)


def _combine_body(pos1_hbm, pos2_hbm, ys_hbm, sh_hbm, out_hbm,
                  p1_v, p2_v, idx1c_v, idx2c_v, r1_v, r2_v, sh_v, out_v,
                  sem0, sem1):
    wid = lax.axis_index("s") * 2 + lax.axis_index("c")
    base = wid * TPW
    pltpu.sync_copy(pos1_hbm.at[pl.ds(base, TPW)], p1_v)
    pltpu.sync_copy(pos2_hbm.at[pl.ds(base, TPW)], p2_v)
    himask = jnp.full((16,), -65536, jnp.int32)

    def chunk(c, carry):
        idx1c_v[...] = p1_v[pl.ds(16 * c, 16)]
        idx2c_v[...] = p2_v[pl.ds(16 * c, 16)]
        cp1 = pltpu.make_async_copy(ys_hbm.at[idx1c_v], r1_v, sem0)
        cp2 = pltpu.make_async_copy(ys_hbm.at[idx2c_v], r2_v, sem1)
        cp1.start()
        cp2.start()
        pltpu.sync_copy(sh_hbm.at[pl.ds(base + 16 * c, 16)], sh_v)
        cp1.wait()
        cp2.wait()

        def row(r, carry2):
            def col(k, carry3):
                sl = pl.ds(16 * k, 16)
                w1 = r1_v[r, sl]
                w2 = r2_v[r, sl]
                wsh = sh_v[r, sl]
                lo = (lax.bitcast_convert_type(w1 << 16, jnp.float32)
                      + lax.bitcast_convert_type(w2 << 16, jnp.float32)
                      + lax.bitcast_convert_type(wsh << 16, jnp.float32))
                hi = (lax.bitcast_convert_type(w1 & himask, jnp.float32)
                      + lax.bitcast_convert_type(w2 & himask, jnp.float32)
                      + lax.bitcast_convert_type(wsh & himask, jnp.float32))
                out_v[r, sl] = lo
                out_v[r, pl.ds(DH + 16 * k, 16)] = hi
                return carry3

            return lax.fori_loop(0, DH // 16, col, carry2)

        lax.fori_loop(0, 16, row, 0)
        pltpu.sync_copy(out_v, out_hbm.at[pl.ds(base + 16 * c, 16)])
        return carry

    lax.fori_loop(0, TPW // 16, chunk, 0)


def _combine(pos1, pos2, ys, sh):
    mesh = plsc.VectorSubcoreMesh(core_axis_name="c", subcore_axis_name="s")
    fn = pl.kernel(
        _combine_body,
        mesh=mesh,
        out_type=jax.ShapeDtypeStruct((T, D), jnp.float32),
        scratch_types=[
            pltpu.VMEM((TPW,), jnp.int32),
            pltpu.VMEM((TPW,), jnp.int32),
            pltpu.VMEM((16,), jnp.int32),
            pltpu.VMEM((16,), jnp.int32),
            pltpu.VMEM((16, DH), jnp.int32),
            pltpu.VMEM((16, DH), jnp.int32),
            pltpu.VMEM((16, DH), jnp.int32),
            pltpu.VMEM((16, D), jnp.float32),
            pltpu.SemaphoreType.DMA,
            pltpu.SemaphoreType.DMA,
        ],
    )
    return fn(pos1, pos2, ys, sh)


def kernel(hidden_states, gate_w, w_gate_up, w_down, ws_gate_up, ws_down):
    b, s, d = hidden_states.shape
    x = hidden_states.reshape(-1, d)

    meta, te_f, wrow1, wrow2, xp = _router(x, gate_w)
    pos1 = meta[:, 0].astype(jnp.int32)
    pos2 = meta[:, 1].astype(jnp.int32)
    te = te_f.reshape(64)[:NT].astype(jnp.int32)
    xs, ws = _dispatch(pos1, pos2, wrow1, wrow2, xp)
    sh = _shared(x, ws_gate_up, ws_down)
    ys = _group_mm(te, xs, ws, w_gate_up, w_down)
    out = _combine(pos1, pos2, ys, sh)
    return out.reshape(b, s, d)


# speedup vs baseline: 1.4646x; 1.0494x over previous
import functools

import jax
import jax.numpy as jnp
from jax import lax
from jax.experimental import pallas as pl
from jax.experimental.pallas import tpu as pltpu
from jax.experimental.pallas import tpu_sc as plsc

T = 2048
D = 1024
DH = 512
FF = 512
E = 8
SHARED_FF2 = 2048
TILE = 256
PAD = 6144
NT = PAD // TILE
BS = 128
NB = 16
NW = 32
TPW = T // NW

def _pack_halves(y_f32):
    yb = y_f32.astype(jnp.bfloat16)
    a = lax.bitcast_convert_type(yb[:, :DH], jnp.uint16).astype(jnp.uint32)
    bhi = lax.bitcast_convert_type(yb[:, DH:], jnp.uint16).astype(jnp.uint32)
    return lax.bitcast_convert_type(a | (bhi << 16), jnp.int32)


def _unpack_halves_bf16(w_i32):
    lo = lax.bitcast_convert_type(w_i32 << 16, jnp.float32)
    hi = lax.bitcast_convert_type(w_i32 & jnp.int32(-65536), jnp.float32)
    return lo.astype(jnp.bfloat16), hi.astype(jnp.bfloat16)


def _router_body(x_ref, gate_ref, meta_ref, te_ref, wrow1_ref, wrow2_ref,
                 xp_ref):
    x = x_ref[...]
    logits = jnp.dot(x, gate_ref[...].T, preferred_element_type=jnp.float32)
    probs = jax.nn.softmax(logits, axis=-1)
    iota_e = lax.broadcasted_iota(jnp.int32, probs.shape, 1)
    m1 = jnp.max(probs, axis=1, keepdims=True)
    idx1 = jnp.min(jnp.where(probs == m1, iota_e, E), axis=1, keepdims=True)
    oh1 = iota_e == idx1
    masked = jnp.where(oh1, -1.0, probs)
    m2 = jnp.max(masked, axis=1, keepdims=True)
    idx2 = jnp.min(jnp.where(masked == m2, iota_e, E), axis=1, keepdims=True)
    oh2 = iota_e == idx2
    wsum = m1 + m2

    sel = (oh1 | oh2).astype(jnp.float32)
    r_i = lax.broadcasted_iota(jnp.int32, (BS, BS), 0)
    c_i = lax.broadcasted_iota(jnp.int32, (BS, BS), 1)
    tril = jnp.where(r_i > c_i, 1.0, 0.0)
    off = jnp.zeros((1, E), jnp.float32)
    ranks = []
    for b in range(NB):
        sb = sel[b * BS:(b + 1) * BS, :]
        ranks.append(jnp.dot(tril, sb, preferred_element_type=jnp.float32)
                     + off)
        off = off + jnp.sum(sb, axis=0, keepdims=True)
    rank = jnp.concatenate(ranks, axis=0)
    counts = off
    cpad = jnp.floor((counts + (TILE - 1.0)) * (1.0 / TILE)) * TILE
    ru = lax.broadcasted_iota(jnp.int32, (E, E), 0)
    cu = lax.broadcasted_iota(jnp.int32, (E, E), 1)
    upper = jnp.where(ru < cu, 1.0, 0.0)
    base = jnp.dot(cpad, upper, preferred_element_type=jnp.float32)
    pos = base + rank
    pos1 = jnp.sum(jnp.where(oh1, pos, 0.0), axis=1, keepdims=True)
    pos2 = jnp.sum(jnp.where(oh2, pos, 0.0), axis=1, keepdims=True)

    col = lax.broadcasted_iota(jnp.int32, (T, 8), 1)
    meta = jnp.where(col == 0, pos1, 0.0)
    meta = jnp.where(col == 1, pos2, meta)
    meta_ref[...] = meta
    wrow1_ref[...] = jnp.broadcast_to(m1 / wsum, (T, 128))
    wrow2_ref[...] = jnp.broadcast_to(m2 / wsum, (T, 128))
    xp_ref[...] = _pack_halves(x)

    incl = base + cpad
    jv = lax.broadcasted_iota(jnp.int32, (64, 1), 0).astype(jnp.float32) * TILE
    teacc = jnp.zeros((64, 1), jnp.float32)
    for e in range(E):
        teacc = teacc + jnp.where(jv >= incl[:, e:e + 1], 1.0, 0.0)
    te_ref[...] = jnp.minimum(teacc, E - 1.0)


def _router(x, gate_w):
    return pl.pallas_call(
        _router_body,
        in_specs=[
            pl.BlockSpec((T, D), lambda: (0, 0)),
            pl.BlockSpec((E, D), lambda: (0, 0)),
        ],
        out_specs=[
            pl.BlockSpec((T, 8), lambda: (0, 0)),
            pl.BlockSpec((64, 1), lambda: (0, 0)),
            pl.BlockSpec((T, 128), lambda: (0, 0)),
            pl.BlockSpec((T, 128), lambda: (0, 0)),
            pl.BlockSpec((T, DH), lambda: (0, 0)),
        ],
        out_shape=[
            jax.ShapeDtypeStruct((T, 8), jnp.float32),
            jax.ShapeDtypeStruct((64, 1), jnp.float32),
            jax.ShapeDtypeStruct((T, 128), jnp.float32),
            jax.ShapeDtypeStruct((T, 128), jnp.float32),
            jax.ShapeDtypeStruct((T, DH), jnp.int32),
        ],
    )(x, gate_w)


def _dispatch_body(pos1_hbm, pos2_hbm, wrow1_hbm, wrow2_hbm, xp_hbm,
                   xs_hbm, ws_hbm, rows_v, wr1_v, wr2_v, idx1_v, idx2_v,
                   sem0, sem1, sem2, sem3):
    wid = lax.axis_index("s") * 2 + lax.axis_index("c")
    base = wid * TPW
    pltpu.sync_copy(pos1_hbm.at[pl.ds(base, TPW)], idx1_v)
    pltpu.sync_copy(pos2_hbm.at[pl.ds(base, TPW)], idx2_v)
    pltpu.sync_copy(xp_hbm.at[pl.ds(base, TPW)], rows_v)
    pltpu.sync_copy(wrow1_hbm.at[pl.ds(base, TPW)], wr1_v)
    pltpu.sync_copy(wrow2_hbm.at[pl.ds(base, TPW)], wr2_v)
    cp1 = pltpu.make_async_copy(rows_v, xs_hbm.at[idx1_v], sem0)
    cp2 = pltpu.make_async_copy(rows_v, xs_hbm.at[idx2_v], sem1)
    cp3 = pltpu.make_async_copy(wr1_v, ws_hbm.at[idx1_v], sem2)
    cp4 = pltpu.make_async_copy(wr2_v, ws_hbm.at[idx2_v], sem3)
    cp1.start()
    cp2.start()
    cp3.start()
    cp4.start()
    cp1.wait()
    cp2.wait()
    cp3.wait()
    cp4.wait()


def _dispatch(pos1, pos2, wrow1, wrow2, xp):
    mesh = plsc.VectorSubcoreMesh(core_axis_name="c", subcore_axis_name="s")
    fn = pl.kernel(
        _dispatch_body,
        mesh=mesh,
        out_type=[
            jax.ShapeDtypeStruct((PAD, DH), jnp.int32),
            jax.ShapeDtypeStruct((PAD, 128), jnp.float32),
        ],
        scratch_types=[
            pltpu.VMEM((TPW, DH), jnp.int32),
            pltpu.VMEM((TPW, 128), jnp.float32),
            pltpu.VMEM((TPW, 128), jnp.float32),
            pltpu.VMEM((TPW,), jnp.int32),
            pltpu.VMEM((TPW,), jnp.int32),
            pltpu.SemaphoreType.DMA,
            pltpu.SemaphoreType.DMA,
            pltpu.SemaphoreType.DMA,
            pltpu.SemaphoreType.DMA,
        ],
    )
    return fn(pos1, pos2, wrow1, wrow2, xp)


def _group_mm_body(te_ref, xs_ref, ws_ref, wgu_ref, wd_ref, ys_ref,
                   wgu_bf, wd_bf):
    i = pl.program_id(0)
    e_now = te_ref[i]
    e_prev = te_ref[jnp.maximum(i - 1, 0)]

    @pl.when(jnp.logical_or(i == 0, e_now != e_prev))
    def _cast():
        wgu_bf[...] = wgu_ref[0].astype(jnp.bfloat16)
        wd_bf[...] = wd_ref[0].astype(jnp.bfloat16)

    a, bh = _unpack_halves_bf16(xs_ref[...])
    gu = (jnp.dot(a, wgu_bf[:, :DH].T, preferred_element_type=jnp.float32)
          + jnp.dot(bh, wgu_bf[:, DH:].T, preferred_element_type=jnp.float32))
    g, u = gu[:, :FF], gu[:, FF:]
    h = (g * jax.nn.sigmoid(g) * u).astype(jnp.bfloat16)
    y = jnp.dot(h, wd_bf[...].T, preferred_element_type=jnp.float32)
    ys_ref[...] = _pack_halves(y * ws_ref[:, :1])


def _group_mm(te, xs, ws, wgu, wd):
    grid_spec = pltpu.PrefetchScalarGridSpec(
        num_scalar_prefetch=1,
        grid=(NT,),
        in_specs=[
            pl.BlockSpec((TILE, DH), lambda i, te: (i, 0)),
            pl.BlockSpec((TILE, 128), lambda i, te: (i, 0)),
            pl.BlockSpec((1, 2 * FF, D), lambda i, te: (te[i], 0, 0)),
            pl.BlockSpec((1, D, FF), lambda i, te: (te[i], 0, 0)),
        ],
        out_specs=pl.BlockSpec((TILE, DH), lambda i, te: (i, 0)),
        scratch_shapes=[
            pltpu.VMEM((2 * FF, D), jnp.bfloat16),
            pltpu.VMEM((D, FF), jnp.bfloat16),
        ],
    )
    return pl.pallas_call(
        _group_mm_body,
        grid_spec=grid_spec,
        out_shape=jax.ShapeDtypeStruct((PAD, DH), jnp.int32),
    )(te, xs, ws, wgu, wd)


def _shared_body(x_ref, wsg_ref, wsd_ref, out_ref, wsg_bf, wsd_bf):
    @pl.when(pl.program_id(0) == 0)
    def _cast():
        wsg_bf[...] = wsg_ref[...].astype(jnp.bfloat16)
        wsd_bf[...] = wsd_ref[...].astype(jnp.bfloat16)

    xb = x_ref[...].astype(jnp.bfloat16)
    sgu = jnp.dot(xb, wsg_bf[...].T, preferred_element_type=jnp.float32)
    sg, su = sgu[:, :SHARED_FF2 // 2], sgu[:, SHARED_FF2 // 2:]
    sh = (sg * jax.nn.sigmoid(sg) * su).astype(jnp.bfloat16)
    out_ref[...] = _pack_halves(
        jnp.dot(sh, wsd_bf[...].T, preferred_element_type=jnp.float32))


def _shared(x, wsg, wsd):
    TM = 256
    return pl.pallas_call(
        _shared_body,
        grid=(T // TM,),
        in_specs=[
            pl.BlockSpec((TM, D), lambda i: (i, 0)),
            pl.BlockSpec((SHARED_FF2, D), lambda i: (0, 0)),
            pl.BlockSpec((D, SHARED_FF2 // 2), lambda i: (0, 0)),
        ],
        out_specs=pl.BlockSpec((TM, DH), lambda i: (i, 0)),
        out_shape=jax.ShapeDtypeStruct((T, DH), jnp.int32),
        scratch_shapes=[
            pltpu.VMEM((SHARED_FF2, D), jnp.bfloat16),
            pltpu.VMEM((D, SHARED_FF2 // 2), jnp.bfloat16),
        ],
    )(x, wsg, wsd)


_NCH = TPW // 16


def _combine_body(pos1_hbm, pos2_hbm, ys_hbm, sh_hbm, out_hbm,
                  p1_v, p2_v,
                  idx1a, idx2a, r1a, r2a, sha, outa, semg_a, semo_a,
                  idx1b, idx2b, r1b, r2b, shb, outb, semg_b, semo_b):
    wid = lax.axis_index("s") * 2 + lax.axis_index("c")
    base = wid * TPW
    pltpu.sync_copy(pos1_hbm.at[pl.ds(base, TPW)], p1_v)
    pltpu.sync_copy(pos2_hbm.at[pl.ds(base, TPW)], p2_v)
    himask = jnp.full((16,), -65536, jnp.int32)
    slots = [(idx1a, idx2a, r1a, r2a, sha, outa, semg_a, semo_a),
             (idx1b, idx2b, r1b, r2b, shb, outb, semg_b, semo_b)]

    def issue(c, s):
        idx1, idx2, r1, r2, shv, _, semg, _ = slots[s]
        idx1[...] = p1_v[pl.ds(16 * c, 16)]
        idx2[...] = p2_v[pl.ds(16 * c, 16)]
        pltpu.make_async_copy(ys_hbm.at[idx1], r1, semg).start()
        pltpu.make_async_copy(ys_hbm.at[idx2], r2, semg).start()
        pltpu.make_async_copy(sh_hbm.at[pl.ds(base + 16 * c, 16)], shv,
                              semg).start()

    issue(0, 0)
    for c in range(_NCH):
        s = c % 2
        idx1, idx2, r1, r2, shv, outv, semg, semo = slots[s]
        if c + 1 < _NCH:
            issue(c + 1, 1 - s)
        pltpu.make_async_copy(ys_hbm.at[idx1], r1, semg).wait()
        pltpu.make_async_copy(ys_hbm.at[idx2], r2, semg).wait()
        pltpu.make_async_copy(sh_hbm.at[pl.ds(base + 16 * c, 16)], shv,
                              semg).wait()
        if c >= 2:
            pltpu.make_async_copy(
                outv, out_hbm.at[pl.ds(base + 16 * (c - 2), 16)],
                semo).wait()

        def row(r, carry2):
            def col(k, carry3):
                sl = pl.ds(16 * k, 16)
                w1 = r1[r, sl]
                w2 = r2[r, sl]
                wsh = shv[r, sl]
                lo = (lax.bitcast_convert_type(w1 << 16, jnp.float32)
                      + lax.bitcast_convert_type(w2 << 16, jnp.float32)
                      + lax.bitcast_convert_type(wsh << 16, jnp.float32))
                hi = (lax.bitcast_convert_type(w1 & himask, jnp.float32)
                      + lax.bitcast_convert_type(w2 & himask, jnp.float32)
                      + lax.bitcast_convert_type(wsh & himask, jnp.float32))
                outv[r, sl] = lo
                outv[r, pl.ds(DH + 16 * k, 16)] = hi
                return carry3

            return lax.fori_loop(0, DH // 16, col, carry2)

        lax.fori_loop(0, 16, row, 0)
        pltpu.make_async_copy(outv, out_hbm.at[pl.ds(base + 16 * c, 16)],
                              semo).start()
    for c in (_NCH - 2, _NCH - 1):
        _, _, _, _, _, outv, _, semo = slots[c % 2]
        pltpu.make_async_copy(outv, out_hbm.at[pl.ds(base + 16 * c, 16)],
                              semo).wait()


def _combine(pos1, pos2, ys, sh):
    mesh = plsc.VectorSubcoreMesh(core_axis_name="c", subcore_axis_name="s")
    slot_scratch = [
        pltpu.VMEM((16,), jnp.int32),
        pltpu.VMEM((16,), jnp.int32),
        pltpu.VMEM((16, DH), jnp.int32),
        pltpu.VMEM((16, DH), jnp.int32),
        pltpu.VMEM((16, DH), jnp.int32),
        pltpu.VMEM((16, D), jnp.float32),
        pltpu.SemaphoreType.DMA,
        pltpu.SemaphoreType.DMA,
    ]
    fn = pl.kernel(
        _combine_body,
        mesh=mesh,
        out_type=jax.ShapeDtypeStruct((T, D), jnp.float32),
        scratch_types=[
            pltpu.VMEM((TPW,), jnp.int32),
            pltpu.VMEM((TPW,), jnp.int32),
        ] + slot_scratch + slot_scratch,
    )
    return fn(pos1, pos2, ys, sh)


def kernel(hidden_states, gate_w, w_gate_up, w_down, ws_gate_up, ws_down):
    b, s, d = hidden_states.shape
    x = hidden_states.reshape(-1, d)

    meta, te_f, wrow1, wrow2, xp = _router(x, gate_w)
    pos1 = meta[:, 0].astype(jnp.int32)
    pos2 = meta[:, 1].astype(jnp.int32)
    te = te_f.reshape(64)[:NT].astype(jnp.int32)
    xs, ws = _dispatch(pos1, pos2, wrow1, wrow2, xp)
    sh = _shared(x, ws_gate_up, ws_down)
    ys = _group_mm(te, xs, ws, w_gate_up, w_down)
    out = _combine(pos1, pos2, ys, sh)
    return out.reshape(b, s, d)


# speedup vs baseline: 1.5254x; 1.0415x over previous
import functools

import jax
import jax.numpy as jnp
from jax import lax
from jax.experimental import pallas as pl
from jax.experimental.pallas import tpu as pltpu
from jax.experimental.pallas import tpu_sc as plsc

T = 2048
D = 1024
DH = 512
FF = 512
E = 8
SHARED_FF2 = 2048
TILE = 256
PAD = 6144
NT = PAD // TILE
BS = 128
NB = 16
NW = 32
TPW = T // NW

def _pack_halves(y_f32):
    yb = y_f32.astype(jnp.bfloat16)
    a = lax.bitcast_convert_type(yb[:, :DH], jnp.uint16).astype(jnp.uint32)
    bhi = lax.bitcast_convert_type(yb[:, DH:], jnp.uint16).astype(jnp.uint32)
    return lax.bitcast_convert_type(a | (bhi << 16), jnp.int32)


def _unpack_halves_bf16(w_i32):
    lo = lax.bitcast_convert_type(w_i32 << 16, jnp.float32)
    hi = lax.bitcast_convert_type(w_i32 & jnp.int32(-65536), jnp.float32)
    return lo.astype(jnp.bfloat16), hi.astype(jnp.bfloat16)


def _router_body(x_ref, gate_ref, meta_ref, te_ref, wrow1_ref, wrow2_ref,
                 xp_ref):
    x = x_ref[...]
    logits = jnp.dot(x, gate_ref[...].T, preferred_element_type=jnp.float32)
    probs = jax.nn.softmax(logits, axis=-1)
    iota_e = lax.broadcasted_iota(jnp.int32, probs.shape, 1)
    m1 = jnp.max(probs, axis=1, keepdims=True)
    idx1 = jnp.min(jnp.where(probs == m1, iota_e, E), axis=1, keepdims=True)
    oh1 = iota_e == idx1
    masked = jnp.where(oh1, -1.0, probs)
    m2 = jnp.max(masked, axis=1, keepdims=True)
    idx2 = jnp.min(jnp.where(masked == m2, iota_e, E), axis=1, keepdims=True)
    oh2 = iota_e == idx2
    wsum = m1 + m2

    sel = (oh1 | oh2).astype(jnp.float32)
    r_i = lax.broadcasted_iota(jnp.int32, (BS, BS), 0)
    c_i = lax.broadcasted_iota(jnp.int32, (BS, BS), 1)
    tril = jnp.where(r_i > c_i, 1.0, 0.0)
    off = jnp.zeros((1, E), jnp.float32)
    ranks = []
    for b in range(NB):
        sb = sel[b * BS:(b + 1) * BS, :]
        ranks.append(jnp.dot(tril, sb, preferred_element_type=jnp.float32)
                     + off)
        off = off + jnp.sum(sb, axis=0, keepdims=True)
    rank = jnp.concatenate(ranks, axis=0)
    counts = off
    cpad = jnp.floor((counts + (TILE - 1.0)) * (1.0 / TILE)) * TILE
    ru = lax.broadcasted_iota(jnp.int32, (E, E), 0)
    cu = lax.broadcasted_iota(jnp.int32, (E, E), 1)
    upper = jnp.where(ru < cu, 1.0, 0.0)
    base = jnp.dot(cpad, upper, preferred_element_type=jnp.float32)
    pos = base + rank
    pos1 = jnp.sum(jnp.where(oh1, pos, 0.0), axis=1, keepdims=True)
    pos2 = jnp.sum(jnp.where(oh2, pos, 0.0), axis=1, keepdims=True)

    col = lax.broadcasted_iota(jnp.int32, (T, 8), 1)
    meta = jnp.where(col == 0, pos1, 0.0)
    meta = jnp.where(col == 1, pos2, meta)
    meta_ref[...] = meta
    wrow1_ref[...] = jnp.broadcast_to(m1 / wsum, (T, 128))
    wrow2_ref[...] = jnp.broadcast_to(m2 / wsum, (T, 128))
    xp_ref[...] = _pack_halves(x)

    incl = base + cpad
    jv = lax.broadcasted_iota(jnp.int32, (64, 1), 0).astype(jnp.float32) * TILE
    teacc = jnp.zeros((64, 1), jnp.float32)
    for e in range(E):
        teacc = teacc + jnp.where(jv >= incl[:, e:e + 1], 1.0, 0.0)
    te = jnp.minimum(teacc, E - 1.0)
    row_i = lax.broadcasted_iota(jnp.int32, (64, 1), 0)
    ntiles = incl[:, E - 1:E] * (1.0 / TILE)
    te_ref[...] = jnp.where(row_i == NT, ntiles, te)


def _router(x, gate_w):
    return pl.pallas_call(
        _router_body,
        in_specs=[
            pl.BlockSpec((T, D), lambda: (0, 0)),
            pl.BlockSpec((E, D), lambda: (0, 0)),
        ],
        out_specs=[
            pl.BlockSpec((T, 8), lambda: (0, 0)),
            pl.BlockSpec((64, 1), lambda: (0, 0)),
            pl.BlockSpec((T, 128), lambda: (0, 0)),
            pl.BlockSpec((T, 128), lambda: (0, 0)),
            pl.BlockSpec((T, DH), lambda: (0, 0)),
        ],
        out_shape=[
            jax.ShapeDtypeStruct((T, 8), jnp.float32),
            jax.ShapeDtypeStruct((64, 1), jnp.float32),
            jax.ShapeDtypeStruct((T, 128), jnp.float32),
            jax.ShapeDtypeStruct((T, 128), jnp.float32),
            jax.ShapeDtypeStruct((T, DH), jnp.int32),
        ],
    )(x, gate_w)


def _dispatch_body(pos1_hbm, pos2_hbm, wrow1_hbm, wrow2_hbm, xp_hbm,
                   xs_hbm, ws_hbm, rows_v, wr1_v, wr2_v, idx1_v, idx2_v,
                   sem0, sem1, sem2, sem3):
    wid = lax.axis_index("s") * 2 + lax.axis_index("c")
    base = wid * TPW
    pltpu.sync_copy(pos1_hbm.at[pl.ds(base, TPW)], idx1_v)
    pltpu.sync_copy(pos2_hbm.at[pl.ds(base, TPW)], idx2_v)
    pltpu.sync_copy(xp_hbm.at[pl.ds(base, TPW)], rows_v)
    pltpu.sync_copy(wrow1_hbm.at[pl.ds(base, TPW)], wr1_v)
    pltpu.sync_copy(wrow2_hbm.at[pl.ds(base, TPW)], wr2_v)
    cp1 = pltpu.make_async_copy(rows_v, xs_hbm.at[idx1_v], sem0)
    cp2 = pltpu.make_async_copy(rows_v, xs_hbm.at[idx2_v], sem1)
    cp3 = pltpu.make_async_copy(wr1_v, ws_hbm.at[idx1_v], sem2)
    cp4 = pltpu.make_async_copy(wr2_v, ws_hbm.at[idx2_v], sem3)
    cp1.start()
    cp2.start()
    cp3.start()
    cp4.start()
    cp1.wait()
    cp2.wait()
    cp3.wait()
    cp4.wait()


def _dispatch(pos1, pos2, wrow1, wrow2, xp):
    mesh = plsc.VectorSubcoreMesh(core_axis_name="c", subcore_axis_name="s")
    fn = pl.kernel(
        _dispatch_body,
        mesh=mesh,
        out_type=[
            jax.ShapeDtypeStruct((PAD, DH), jnp.int32),
            jax.ShapeDtypeStruct((PAD, 128), jnp.float32),
        ],
        scratch_types=[
            pltpu.VMEM((TPW, DH), jnp.int32),
            pltpu.VMEM((TPW, 128), jnp.float32),
            pltpu.VMEM((TPW, 128), jnp.float32),
            pltpu.VMEM((TPW,), jnp.int32),
            pltpu.VMEM((TPW,), jnp.int32),
            pltpu.SemaphoreType.DMA,
            pltpu.SemaphoreType.DMA,
            pltpu.SemaphoreType.DMA,
            pltpu.SemaphoreType.DMA,
        ],
    )
    return fn(pos1, pos2, wrow1, wrow2, xp)


def _group_mm_body(te_ref, xs_ref, ws_ref, wgu_ref, wd_ref, ys_ref,
                   wgu_bf, wd_bf):
    i = pl.program_id(0)
    e_now = te_ref[i]
    e_prev = te_ref[jnp.maximum(i - 1, 0)]

    @pl.when(i < te_ref[NT])
    def _live():
        @pl.when(jnp.logical_or(i == 0, e_now != e_prev))
        def _cast():
            wgu_bf[...] = wgu_ref[0].astype(jnp.bfloat16)
            wd_bf[...] = wd_ref[0].astype(jnp.bfloat16)

        a, bh = _unpack_halves_bf16(xs_ref[...])
        gu = (jnp.dot(a, wgu_bf[:, :DH].T,
                      preferred_element_type=jnp.float32)
              + jnp.dot(bh, wgu_bf[:, DH:].T,
                        preferred_element_type=jnp.float32))
        g, u = gu[:, :FF], gu[:, FF:]
        h = (g * jax.nn.sigmoid(g) * u).astype(jnp.bfloat16)
        y = jnp.dot(h, wd_bf[...].T, preferred_element_type=jnp.float32)
        ys_ref[...] = _pack_halves(y * ws_ref[:, :1])


def _group_mm(te, xs, ws, wgu, wd):
    grid_spec = pltpu.PrefetchScalarGridSpec(
        num_scalar_prefetch=1,
        grid=(NT,),
        in_specs=[
            pl.BlockSpec((TILE, DH), lambda i, te: (i, 0)),
            pl.BlockSpec((TILE, 128), lambda i, te: (i, 0)),
            pl.BlockSpec((1, 2 * FF, D), lambda i, te: (te[i], 0, 0)),
            pl.BlockSpec((1, D, FF), lambda i, te: (te[i], 0, 0)),
        ],
        out_specs=pl.BlockSpec((TILE, DH), lambda i, te: (i, 0)),
        scratch_shapes=[
            pltpu.VMEM((2 * FF, D), jnp.bfloat16),
            pltpu.VMEM((D, FF), jnp.bfloat16),
        ],
    )
    return pl.pallas_call(
        _group_mm_body,
        grid_spec=grid_spec,
        out_shape=jax.ShapeDtypeStruct((PAD, DH), jnp.int32),
    )(te, xs, ws, wgu, wd)


def _shared_body(x_ref, wsg_ref, wsd_ref, out_ref, wsg_bf, wsd_bf):
    @pl.when(pl.program_id(0) == 0)
    def _cast():
        wsg_bf[...] = wsg_ref[...].astype(jnp.bfloat16)
        wsd_bf[...] = wsd_ref[...].astype(jnp.bfloat16)

    xb = x_ref[...].astype(jnp.bfloat16)
    sgu = jnp.dot(xb, wsg_bf[...].T, preferred_element_type=jnp.float32)
    sg, su = sgu[:, :SHARED_FF2 // 2], sgu[:, SHARED_FF2 // 2:]
    sh = (sg * jax.nn.sigmoid(sg) * su).astype(jnp.bfloat16)
    out_ref[...] = _pack_halves(
        jnp.dot(sh, wsd_bf[...].T, preferred_element_type=jnp.float32))


def _shared(x, wsg, wsd):
    TM = 512
    return pl.pallas_call(
        _shared_body,
        grid=(T // TM,),
        in_specs=[
            pl.BlockSpec((TM, D), lambda i: (i, 0)),
            pl.BlockSpec((SHARED_FF2, D), lambda i: (0, 0)),
            pl.BlockSpec((D, SHARED_FF2 // 2), lambda i: (0, 0)),
        ],
        out_specs=pl.BlockSpec((TM, DH), lambda i: (i, 0)),
        out_shape=jax.ShapeDtypeStruct((T, DH), jnp.int32),
        scratch_shapes=[
            pltpu.VMEM((SHARED_FF2, D), jnp.bfloat16),
            pltpu.VMEM((D, SHARED_FF2 // 2), jnp.bfloat16),
        ],
    )(x, wsg, wsd)


_NCH = TPW // 16


def _combine_body(pos1_hbm, pos2_hbm, ys_hbm, sh_hbm, out_hbm,
                  p1_v, p2_v,
                  idx1a, idx2a, r1a, r2a, sha, outa, semg_a, semo_a,
                  idx1b, idx2b, r1b, r2b, shb, outb, semg_b, semo_b):
    wid = lax.axis_index("s") * 2 + lax.axis_index("c")
    base = wid * TPW
    pltpu.sync_copy(pos1_hbm.at[pl.ds(base, TPW)], p1_v)
    pltpu.sync_copy(pos2_hbm.at[pl.ds(base, TPW)], p2_v)
    himask = jnp.full((16,), -65536, jnp.int32)
    slots = [(idx1a, idx2a, r1a, r2a, sha, outa, semg_a, semo_a),
             (idx1b, idx2b, r1b, r2b, shb, outb, semg_b, semo_b)]

    def issue(c, s):
        idx1, idx2, r1, r2, shv, _, semg, _ = slots[s]
        idx1[...] = p1_v[pl.ds(16 * c, 16)]
        idx2[...] = p2_v[pl.ds(16 * c, 16)]
        pltpu.make_async_copy(ys_hbm.at[idx1], r1, semg).start()
        pltpu.make_async_copy(ys_hbm.at[idx2], r2, semg).start()
        pltpu.make_async_copy(sh_hbm.at[pl.ds(base + 16 * c, 16)], shv,
                              semg).start()

    issue(0, 0)
    for c in range(_NCH):
        s = c % 2
        idx1, idx2, r1, r2, shv, outv, semg, semo = slots[s]
        if c + 1 < _NCH:
            issue(c + 1, 1 - s)
        pltpu.make_async_copy(ys_hbm.at[idx1], r1, semg).wait()
        pltpu.make_async_copy(ys_hbm.at[idx2], r2, semg).wait()
        pltpu.make_async_copy(sh_hbm.at[pl.ds(base + 16 * c, 16)], shv,
                              semg).wait()
        if c >= 2:
            pltpu.make_async_copy(
                outv, out_hbm.at[pl.ds(base + 16 * (c - 2), 16)],
                semo).wait()

        def row(r, carry2):
            def col(k, carry3):
                sl = pl.ds(16 * k, 16)
                w1 = r1[r, sl]
                w2 = r2[r, sl]
                wsh = shv[r, sl]
                lo = (lax.bitcast_convert_type(w1 << 16, jnp.float32)
                      + lax.bitcast_convert_type(w2 << 16, jnp.float32)
                      + lax.bitcast_convert_type(wsh << 16, jnp.float32))
                hi = (lax.bitcast_convert_type(w1 & himask, jnp.float32)
                      + lax.bitcast_convert_type(w2 & himask, jnp.float32)
                      + lax.bitcast_convert_type(wsh & himask, jnp.float32))
                outv[r, sl] = lo
                outv[r, pl.ds(DH + 16 * k, 16)] = hi
                return carry3

            return lax.fori_loop(0, DH // 16, col, carry2)

        lax.fori_loop(0, 16, row, 0)
        pltpu.make_async_copy(outv, out_hbm.at[pl.ds(base + 16 * c, 16)],
                              semo).start()
    for c in (_NCH - 2, _NCH - 1):
        _, _, _, _, _, outv, _, semo = slots[c % 2]
        pltpu.make_async_copy(outv, out_hbm.at[pl.ds(base + 16 * c, 16)],
                              semo).wait()


def _combine(pos1, pos2, ys, sh):
    mesh = plsc.VectorSubcoreMesh(core_axis_name="c", subcore_axis_name="s")
    slot_scratch = [
        pltpu.VMEM((16,), jnp.int32),
        pltpu.VMEM((16,), jnp.int32),
        pltpu.VMEM((16, DH), jnp.int32),
        pltpu.VMEM((16, DH), jnp.int32),
        pltpu.VMEM((16, DH), jnp.int32),
        pltpu.VMEM((16, D), jnp.float32),
        pltpu.SemaphoreType.DMA,
        pltpu.SemaphoreType.DMA,
    ]
    fn = pl.kernel(
        _combine_body,
        mesh=mesh,
        out_type=jax.ShapeDtypeStruct((T, D), jnp.float32),
        scratch_types=[
            pltpu.VMEM((TPW,), jnp.int32),
            pltpu.VMEM((TPW,), jnp.int32),
        ] + slot_scratch + slot_scratch,
    )
    return fn(pos1, pos2, ys, sh)


def kernel(hidden_states, gate_w, w_gate_up, w_down, ws_gate_up, ws_down):
    b, s, d = hidden_states.shape
    x = hidden_states.reshape(-1, d)

    meta, te_f, wrow1, wrow2, xp = _router(x, gate_w)
    pos1 = meta[:, 0].astype(jnp.int32)
    pos2 = meta[:, 1].astype(jnp.int32)
    te = te_f.reshape(64)[:NT + 1].astype(jnp.int32)
    xs, ws = _dispatch(pos1, pos2, wrow1, wrow2, xp)
    sh = _shared(x, ws_gate_up, ws_down)
    ys = _group_mm(te, xs, ws, w_gate_up, w_down)
    out = _combine(pos1, pos2, ys, sh)
    return out.reshape(b, s, d)


# speedup vs baseline: 1.5302x; 1.0031x over previous
import functools

import jax
import jax.numpy as jnp
from jax import lax
from jax.experimental import pallas as pl
from jax.experimental.pallas import tpu as pltpu
from jax.experimental.pallas import tpu_sc as plsc

T = 2048
D = 1024
DH = 512
FF = 512
E = 8
SHARED_FF2 = 2048
TILE = 256
PAD = 6144
NT = PAD // TILE
BS = 128
NB = 16
NW = 32
TPW = T // NW

def _pack_halves(y_f32):
    yb = y_f32.astype(jnp.bfloat16)
    a = lax.bitcast_convert_type(yb[:, :DH], jnp.uint16).astype(jnp.uint32)
    bhi = lax.bitcast_convert_type(yb[:, DH:], jnp.uint16).astype(jnp.uint32)
    return lax.bitcast_convert_type(a | (bhi << 16), jnp.int32)


def _unpack_halves_bf16(w_i32):
    lo = lax.bitcast_convert_type(w_i32 << 16, jnp.float32)
    hi = lax.bitcast_convert_type(w_i32 & jnp.int32(-65536), jnp.float32)
    return lo.astype(jnp.bfloat16), hi.astype(jnp.bfloat16)


def _router_body(x_ref, gate_ref, meta_ref, te_ref, wrow1_ref, wrow2_ref,
                 xp_ref):
    x = x_ref[...]
    logits = jnp.dot(x, gate_ref[...].T, preferred_element_type=jnp.float32)
    probs = jax.nn.softmax(logits, axis=-1)
    iota_e = lax.broadcasted_iota(jnp.int32, probs.shape, 1)
    m1 = jnp.max(probs, axis=1, keepdims=True)
    idx1 = jnp.min(jnp.where(probs == m1, iota_e, E), axis=1, keepdims=True)
    oh1 = iota_e == idx1
    masked = jnp.where(oh1, -1.0, probs)
    m2 = jnp.max(masked, axis=1, keepdims=True)
    idx2 = jnp.min(jnp.where(masked == m2, iota_e, E), axis=1, keepdims=True)
    oh2 = iota_e == idx2
    wsum = m1 + m2

    sel = (oh1 | oh2).astype(jnp.float32)
    r_i = lax.broadcasted_iota(jnp.int32, (BS, BS), 0)
    c_i = lax.broadcasted_iota(jnp.int32, (BS, BS), 1)
    tril = jnp.where(r_i > c_i, 1.0, 0.0)
    off = jnp.zeros((1, E), jnp.float32)
    ranks = []
    for b in range(NB):
        sb = sel[b * BS:(b + 1) * BS, :]
        ranks.append(jnp.dot(tril, sb, preferred_element_type=jnp.float32)
                     + off)
        off = off + jnp.sum(sb, axis=0, keepdims=True)
    rank = jnp.concatenate(ranks, axis=0)
    counts = off
    cpad = jnp.floor((counts + (TILE - 1.0)) * (1.0 / TILE)) * TILE
    ru = lax.broadcasted_iota(jnp.int32, (E, E), 0)
    cu = lax.broadcasted_iota(jnp.int32, (E, E), 1)
    upper = jnp.where(ru < cu, 1.0, 0.0)
    base = jnp.dot(cpad, upper, preferred_element_type=jnp.float32)
    pos = base + rank
    pos1 = jnp.sum(jnp.where(oh1, pos, 0.0), axis=1, keepdims=True)
    pos2 = jnp.sum(jnp.where(oh2, pos, 0.0), axis=1, keepdims=True)

    col = lax.broadcasted_iota(jnp.int32, (T, 8), 1)
    meta = jnp.where(col == 0, pos1, 0.0)
    meta = jnp.where(col == 1, pos2, meta)
    meta_ref[...] = meta
    wrow1_ref[...] = jnp.broadcast_to(m1 / wsum, (T, 128))
    wrow2_ref[...] = jnp.broadcast_to(m2 / wsum, (T, 128))
    xp_ref[...] = _pack_halves(x)

    incl = base + cpad
    jv = lax.broadcasted_iota(jnp.int32, (64, 1), 0).astype(jnp.float32) * TILE
    teacc = jnp.zeros((64, 1), jnp.float32)
    for e in range(E):
        teacc = teacc + jnp.where(jv >= incl[:, e:e + 1], 1.0, 0.0)
    te = jnp.minimum(teacc, E - 1.0)
    row_i = lax.broadcasted_iota(jnp.int32, (64, 1), 0)
    ntiles = incl[:, E - 1:E] * (1.0 / TILE)
    te_ref[...] = jnp.where(row_i == NT, ntiles, te)


def _router(x, gate_w):
    return pl.pallas_call(
        _router_body,
        in_specs=[
            pl.BlockSpec((T, D), lambda: (0, 0)),
            pl.BlockSpec((E, D), lambda: (0, 0)),
        ],
        out_specs=[
            pl.BlockSpec((T, 8), lambda: (0, 0)),
            pl.BlockSpec((64, 1), lambda: (0, 0)),
            pl.BlockSpec((T, 128), lambda: (0, 0)),
            pl.BlockSpec((T, 128), lambda: (0, 0)),
            pl.BlockSpec((T, DH), lambda: (0, 0)),
        ],
        out_shape=[
            jax.ShapeDtypeStruct((T, 8), jnp.float32),
            jax.ShapeDtypeStruct((64, 1), jnp.float32),
            jax.ShapeDtypeStruct((T, 128), jnp.float32),
            jax.ShapeDtypeStruct((T, 128), jnp.float32),
            jax.ShapeDtypeStruct((T, DH), jnp.int32),
        ],
    )(x, gate_w)


def _dispatch_body(pos1_hbm, pos2_hbm, wrow1_hbm, wrow2_hbm, xp_hbm,
                   xs_hbm, ws_hbm, rows_v, wr1_v, wr2_v, idx1_v, idx2_v,
                   sem0, sem1, sem2, sem3):
    wid = lax.axis_index("s") * 2 + lax.axis_index("c")
    base = wid * TPW
    pltpu.sync_copy(pos1_hbm.at[pl.ds(base, TPW)], idx1_v)
    pltpu.sync_copy(pos2_hbm.at[pl.ds(base, TPW)], idx2_v)
    pltpu.sync_copy(xp_hbm.at[pl.ds(base, TPW)], rows_v)
    pltpu.sync_copy(wrow1_hbm.at[pl.ds(base, TPW)], wr1_v)
    pltpu.sync_copy(wrow2_hbm.at[pl.ds(base, TPW)], wr2_v)
    cp1 = pltpu.make_async_copy(rows_v, xs_hbm.at[idx1_v], sem0)
    cp2 = pltpu.make_async_copy(rows_v, xs_hbm.at[idx2_v], sem1)
    cp3 = pltpu.make_async_copy(wr1_v, ws_hbm.at[idx1_v], sem2)
    cp4 = pltpu.make_async_copy(wr2_v, ws_hbm.at[idx2_v], sem3)
    cp1.start()
    cp2.start()
    cp3.start()
    cp4.start()
    cp1.wait()
    cp2.wait()
    cp3.wait()
    cp4.wait()


def _dispatch(pos1, pos2, wrow1, wrow2, xp):
    mesh = plsc.VectorSubcoreMesh(core_axis_name="c", subcore_axis_name="s")
    fn = pl.kernel(
        _dispatch_body,
        mesh=mesh,
        out_type=[
            jax.ShapeDtypeStruct((PAD, DH), jnp.int32),
            jax.ShapeDtypeStruct((PAD, 128), jnp.float32),
        ],
        scratch_types=[
            pltpu.VMEM((TPW, DH), jnp.int32),
            pltpu.VMEM((TPW, 128), jnp.float32),
            pltpu.VMEM((TPW, 128), jnp.float32),
            pltpu.VMEM((TPW,), jnp.int32),
            pltpu.VMEM((TPW,), jnp.int32),
            pltpu.SemaphoreType.DMA,
            pltpu.SemaphoreType.DMA,
            pltpu.SemaphoreType.DMA,
            pltpu.SemaphoreType.DMA,
        ],
    )
    return fn(pos1, pos2, wrow1, wrow2, xp)


def _group_mm_body(te_ref, xs_ref, ws_ref, wgu_ref, wd_ref, ys_ref,
                   wgu_bf, wd_bf):
    i = pl.program_id(0)
    e_now = te_ref[i]
    e_prev = te_ref[jnp.maximum(i - 1, 0)]

    @pl.when(i < te_ref[NT])
    def _live():
        @pl.when(jnp.logical_or(i == 0, e_now != e_prev))
        def _cast():
            wgu_bf[...] = wgu_ref[0].astype(jnp.bfloat16)
            wd_bf[...] = wd_ref[0].astype(jnp.bfloat16)

        a, bh = _unpack_halves_bf16(xs_ref[...])
        gu = (jnp.dot(a, wgu_bf[:, :DH].T,
                      preferred_element_type=jnp.float32)
              + jnp.dot(bh, wgu_bf[:, DH:].T,
                        preferred_element_type=jnp.float32))
        g, u = gu[:, :FF], gu[:, FF:]
        h = (g * jax.nn.sigmoid(g) * u).astype(jnp.bfloat16)
        y = jnp.dot(h, wd_bf[...].T, preferred_element_type=jnp.float32)
        ys_ref[...] = _pack_halves(y * ws_ref[:, :1])


def _group_mm(te, xs, ws, wgu, wd):
    grid_spec = pltpu.PrefetchScalarGridSpec(
        num_scalar_prefetch=1,
        grid=(NT,),
        in_specs=[
            pl.BlockSpec((TILE, DH), lambda i, te: (i, 0)),
            pl.BlockSpec((TILE, 128), lambda i, te: (i, 0)),
            pl.BlockSpec((1, 2 * FF, D), lambda i, te: (te[i], 0, 0)),
            pl.BlockSpec((1, D, FF), lambda i, te: (te[i], 0, 0)),
        ],
        out_specs=pl.BlockSpec((TILE, DH), lambda i, te: (i, 0)),
        scratch_shapes=[
            pltpu.VMEM((2 * FF, D), jnp.bfloat16),
            pltpu.VMEM((D, FF), jnp.bfloat16),
        ],
    )
    return pl.pallas_call(
        _group_mm_body,
        grid_spec=grid_spec,
        out_shape=jax.ShapeDtypeStruct((PAD, DH), jnp.int32),
    )(te, xs, ws, wgu, wd)


def _shared_body(x_ref, wsg_ref, wsd_ref, out_ref, wsg_bf, wsd_bf):
    @pl.when(pl.program_id(0) == 0)
    def _cast():
        wsg_bf[...] = wsg_ref[...].astype(jnp.bfloat16)
        wsd_bf[...] = wsd_ref[...].astype(jnp.bfloat16)

    xb = x_ref[...].astype(jnp.bfloat16)
    sgu = jnp.dot(xb, wsg_bf[...].T, preferred_element_type=jnp.float32)
    sg, su = sgu[:, :SHARED_FF2 // 2], sgu[:, SHARED_FF2 // 2:]
    sh = (sg * jax.nn.sigmoid(sg) * su).astype(jnp.bfloat16)
    out_ref[...] = _pack_halves(
        jnp.dot(sh, wsd_bf[...].T, preferred_element_type=jnp.float32))


def _shared(x, wsg, wsd):
    TM = 512
    return pl.pallas_call(
        _shared_body,
        grid=(T // TM,),
        in_specs=[
            pl.BlockSpec((TM, D), lambda i: (i, 0)),
            pl.BlockSpec((SHARED_FF2, D), lambda i: (0, 0)),
            pl.BlockSpec((D, SHARED_FF2 // 2), lambda i: (0, 0)),
        ],
        out_specs=pl.BlockSpec((TM, DH), lambda i: (i, 0)),
        out_shape=jax.ShapeDtypeStruct((T, DH), jnp.int32),
        scratch_shapes=[
            pltpu.VMEM((SHARED_FF2, D), jnp.bfloat16),
            pltpu.VMEM((D, SHARED_FF2 // 2), jnp.bfloat16),
        ],
    )(x, wsg, wsd)


_NCH = TPW // 16


def _combine_body(pos1_hbm, pos2_hbm, ys_hbm, sh_hbm, out_hbm,
                  p1_v, p2_v,
                  idx1a, idx2a, r1a, r2a, sha, outa, semg_a, semo_a,
                  idx1b, idx2b, r1b, r2b, shb, outb, semg_b, semo_b):
    wid = lax.axis_index("s") * 2 + lax.axis_index("c")
    base = wid * TPW
    pltpu.sync_copy(pos1_hbm.at[pl.ds(base, TPW)], p1_v)
    pltpu.sync_copy(pos2_hbm.at[pl.ds(base, TPW)], p2_v)
    himask = jnp.full((16,), -65536, jnp.int32)
    slots = [(idx1a, idx2a, r1a, r2a, sha, outa, semg_a, semo_a),
             (idx1b, idx2b, r1b, r2b, shb, outb, semg_b, semo_b)]

    def issue(c, s):
        idx1, idx2, r1, r2, shv, _, semg, _ = slots[s]
        idx1[...] = p1_v[pl.ds(16 * c, 16)]
        idx2[...] = p2_v[pl.ds(16 * c, 16)]
        pltpu.make_async_copy(ys_hbm.at[idx1], r1, semg).start()
        pltpu.make_async_copy(ys_hbm.at[idx2], r2, semg).start()
        pltpu.make_async_copy(sh_hbm.at[pl.ds(base + 16 * c, 16)], shv,
                              semg).start()

    issue(0, 0)
    for c in range(_NCH):
        s = c % 2
        idx1, idx2, r1, r2, shv, outv, semg, semo = slots[s]
        if c + 1 < _NCH:
            issue(c + 1, 1 - s)
        pltpu.make_async_copy(ys_hbm.at[idx1], r1, semg).wait()
        pltpu.make_async_copy(ys_hbm.at[idx2], r2, semg).wait()
        pltpu.make_async_copy(sh_hbm.at[pl.ds(base + 16 * c, 16)], shv,
                              semg).wait()
        if c >= 2:
            pltpu.make_async_copy(
                outv, out_hbm.at[pl.ds(base + 16 * (c - 2), 16)],
                semo).wait()

        def row(r, carry2):
            def col(k, carry3):
                sl = pl.ds(16 * k, 16)
                w1 = r1[r, sl]
                w2 = r2[r, sl]
                wsh = shv[r, sl]
                lo = (lax.bitcast_convert_type(w1 << 16, jnp.float32)
                      + lax.bitcast_convert_type(w2 << 16, jnp.float32)
                      + lax.bitcast_convert_type(wsh << 16, jnp.float32))
                hi = (lax.bitcast_convert_type(w1 & himask, jnp.float32)
                      + lax.bitcast_convert_type(w2 & himask, jnp.float32)
                      + lax.bitcast_convert_type(wsh & himask, jnp.float32))
                outv[r, sl] = lo
                outv[r, pl.ds(DH + 16 * k, 16)] = hi
                return carry3

            return lax.fori_loop(0, DH // 16, col, carry2)

        lax.fori_loop(0, 16, row, 0)
        pltpu.make_async_copy(outv, out_hbm.at[pl.ds(base + 16 * c, 16)],
                              semo).start()
    for c in (_NCH - 2, _NCH - 1):
        _, _, _, _, _, outv, _, semo = slots[c % 2]
        pltpu.make_async_copy(outv, out_hbm.at[pl.ds(base + 16 * c, 16)],
                              semo).wait()


def _combine(pos1, pos2, ys, sh):
    mesh = plsc.VectorSubcoreMesh(core_axis_name="c", subcore_axis_name="s")
    slot_scratch = [
        pltpu.VMEM((16,), jnp.int32),
        pltpu.VMEM((16,), jnp.int32),
        pltpu.VMEM((16, DH), jnp.int32),
        pltpu.VMEM((16, DH), jnp.int32),
        pltpu.VMEM((16, DH), jnp.int32),
        pltpu.VMEM((16, D), jnp.float32),
        pltpu.SemaphoreType.DMA,
        pltpu.SemaphoreType.DMA,
    ]
    fn = pl.kernel(
        _combine_body,
        mesh=mesh,
        out_type=jax.ShapeDtypeStruct((T, D), jnp.float32),
        scratch_types=[
            pltpu.VMEM((TPW,), jnp.int32),
            pltpu.VMEM((TPW,), jnp.int32),
        ] + slot_scratch + slot_scratch,
    )
    return fn(pos1, pos2, ys, sh)


def kernel(hidden_states, gate_w, w_gate_up, w_down, ws_gate_up, ws_down):
    b, s, d = hidden_states.shape
    x = hidden_states.reshape(-1, d)

    meta, te_f, wrow1, wrow2, xp = _router(x, gate_w)
    pos1 = meta[:, 0].astype(jnp.int32)
    pos2 = meta[:, 1].astype(jnp.int32)
    te = te_f.reshape(64)[:NT + 1].astype(jnp.int32)
    xs, ws = _dispatch(pos1, pos2, wrow1, wrow2, xp)
    sh = _shared(x, ws_gate_up, ws_down)
    ys = _group_mm(te, xs, ws, w_gate_up, w_down)
    out = _combine(pos1, pos2, ys, sh)
    out = out + 0.0
    return out.reshape(b, s, d)
